# baseline (speedup 1.0000x reference)
import functools
import math

import jax
import jax.numpy as jnp
from jax.experimental import pallas as pl

HID = 64
MAXSTEP = 2
BB = 4
BF = jnp.bfloat16
F32 = jnp.float32


def _unpair(x2):
    half = x2.shape[1] // 2
    lo = x2[:, :half]
    hi = x2[:, half:]
    return jnp.concatenate([lo[:, None, :], hi[:, None, :]], axis=1).reshape(
        x2.shape[0] * 2, half)


def _attention(st2b, enc_ref, folde_ref, exp12_ref, foldh_ref, j,
               t_len, scale):
    ec = jnp.concatenate(
        [enc_ref[j, t].astype(BF) for t in range(t_len)], axis=1)
    stt = jnp.concatenate([st2b] * t_len, axis=1)
    m = stt * ec
    e = jnp.dot(m, folde_ref[...], preferred_element_type=F32)
    e = e * (1.0 / scale)
    ps = []
    for p in range(2):
        ep = e[:, p * t_len:(p + 1) * t_len]
        mx = jnp.max(ep, axis=1, keepdims=True)
        w = jnp.exp(ep - mx)
        den = jnp.sum(w, axis=1, keepdims=True)
        ps.append(w / den)
    p2 = jnp.concatenate(ps, axis=1).astype(BF)
    pex = jnp.dot(p2, exp12_ref[...], preferred_element_type=F32).astype(BF)
    wc = pex * ec
    w2 = jnp.dot(wc, foldh_ref[...], preferred_element_type=F32)
    return w2.astype(BF)


def _mega_kernel(state_ref, enc_ref, inp_ref, s2_ref,
                 folde_ref, exp12_ref, foldh_ref,
                 wg_ref, bg_ref, wc_ref, bc_ref,
                 p1_ref, p1b_ref, p2_ref, p2b_ref, o_ref,
                 *, t_len, scale, in_dim, n):
    stfs, stbs, wts = [], [], []
    for j in range(BB):
        st2 = state_ref[j]
        stf = _unpair(st2)
        stfs.append(stf)
        stbs.append(stf.astype(BF))
        w2 = _attention(st2.astype(BF), enc_ref, folde_ref, exp12_ref,
                        foldh_ref, j, t_len, scale)
        wts.append(_unpair(w2))

    ibs = [inp_ref[0][:, j * in_dim:(j + 1) * in_dim].astype(BF)
           for j in range(BB)]
    xc = jnp.concatenate(ibs + wts + stbs, axis=1)

    chains = []
    for s in range(2):
        x1 = jnp.dot(s2_ref[s], xc, preferred_element_type=F32).astype(BF)
        x2 = jnp.dot(s2_ref[s], x1, preferred_element_type=F32).astype(BF)
        chains += [x1, x2]

    oi = lambda j: j * in_dim
    ow = lambda j: BB * in_dim + j * HID
    osn = lambda j: BB * in_dim + BB * HID + j * HID

    def trio(arr, j):
        return [arr[:, oi(j):oi(j) + in_dim],
                arr[:, ow(j):ow(j) + HID],
                arr[:, osn(j):osn(j) + HID]]

    gs = []
    for j in range(BB):
        cols = trio(xc, j)
        for ch in chains:
            cols += trio(ch, j)
        xj = jnp.concatenate(cols, axis=1)
        gs.append(jax.nn.sigmoid(
            jnp.dot(xj, wg_ref[...], preferred_element_type=F32) + bg_ref[...]))

    rss = [(gs[j][:, :HID] * stfs[j]).astype(BF) for j in range(BB)]
    xr = jnp.concatenate(rss, axis=1)
    rchains = []
    for s in range(2):
        x1 = jnp.dot(s2_ref[s], xr, preferred_element_type=F32).astype(BF)
        x2 = jnp.dot(s2_ref[s], x1, preferred_element_type=F32).astype(BF)
        rchains += [x1, x2]

    for j in range(BB):
        cols = trio(xc, j)[:2] + [xr[:, j * HID:(j + 1) * HID]]
        for ch, rch in zip(chains, rchains):
            cols += trio(ch, j)[:2] + [rch[:, j * HID:(j + 1) * HID]]
        xj = jnp.concatenate(cols, axis=1)
        c = jnp.tanh(
            jnp.dot(xj, wc_ref[...], preferred_element_type=F32) + bc_ref[...])
        u = gs[j][:, HID:]
        ns = u * stfs[j] + (1.0 - u) * c
        h1 = jnp.dot(ns.astype(BF), p1_ref[...],
                     preferred_element_type=F32) + p1b_ref[...]
        h1 = jnp.maximum(h1, 0.0)
        o_ref[j] = jnp.dot(h1.astype(BF), p2_ref[...],
                           preferred_element_type=F32) + p2b_ref[...]


def kernel(support0, support1, inputs, state, encoder_outputs,
           W_gate, b_gate, W_cand, b_cand, P1, p1b, P2, p2b):
    b, n, in_dim = inputs.shape
    t_len = encoder_outputs.shape[1]
    h = state.shape[2]
    scale = math.sqrt(float(n * h))

    s2 = jnp.stack([support0, support1]).astype(BF)
    enc2 = encoder_outputs.reshape(b, t_len, n // 2, 2 * h)
    state2 = state.reshape(b, n // 2, 2 * h)
    i0 = inputs.transpose(1, 0, 2).reshape(n, b * in_dim)
    i0 = i0.reshape(n, b // BB, BB * in_dim).transpose(1, 0, 2)

    th2 = t_len * 2 * h
    lane = jnp.arange(th2)
    t_idx = lane // (2 * h)
    par = (lane % (2 * h)) // h
    q = jnp.arange(2 * t_len)
    folde = ((t_idx[:, None] == (q % t_len)[None, :])
             & (par[:, None] == (q // t_len)[None, :])).astype(BF)
    exp12 = folde.T
    foldh = ((lane % (2 * h))[:, None] == jnp.arange(2 * h)[None, :]).astype(BF)

    in_size = in_dim + 2 * h
    nm = 2 * MAXSTEP + 1
    wg2 = W_gate.reshape(in_size, nm, 2 * h).transpose(1, 0, 2).reshape(in_size * nm, 2 * h).astype(BF)
    wc2 = W_cand.reshape(in_size, nm, h).transpose(1, 0, 2).reshape(in_size * nm, h).astype(BF)

    full = lambda shp: pl.BlockSpec(shp, lambda i: tuple(0 for _ in shp))

    out = pl.pallas_call(
        functools.partial(_mega_kernel, t_len=t_len, scale=scale,
                          in_dim=in_dim, n=n),
        grid=(b // BB,),
        in_specs=[
            pl.BlockSpec((BB, n // 2, 2 * h), lambda i: (i, 0, 0)),
            pl.BlockSpec((BB, t_len, n // 2, 2 * h), lambda i: (i, 0, 0, 0)),
            pl.BlockSpec((1, n, BB * in_dim), lambda i: (i, 0, 0)),
            full((2, n, n)),
            full((th2, 2 * t_len)), full((2 * t_len, th2)), full((th2, 2 * h)),
            full((in_size * nm, 2 * h)), full((1, 2 * h)),
            full((in_size * nm, h)), full((1, h)),
            full((h, h)), full((1, h)), full((h, 1)), full((1, 1)),
        ],
        out_specs=pl.BlockSpec((BB, n, 1), lambda i: (i, 0, 0)),
        out_shape=jax.ShapeDtypeStruct((b, n, 1), F32),
    )(state2, enc2, i0, s2, folde, exp12, foldh,
      wg2, b_gate.reshape(1, 2 * h), wc2, b_cand.reshape(1, h),
      P1.astype(BF), p1b.reshape(1, h), P2.astype(BF), p2b.reshape(1, 1))

    return out.reshape(b, n)


# speedup vs baseline: 1.1965x; 1.1965x over previous
import functools
import math

import jax
import jax.numpy as jnp
from jax.experimental import pallas as pl

HID = 64
MAXSTEP = 2
BB = 2
BF = jnp.bfloat16
F32 = jnp.float32


def _attention(state_ref, enc_ref, folde_ref, exp12_ref, foldh_ref, j,
               t_len, scale):
    ec = jnp.concatenate(
        [enc_ref[j, t].astype(BF) for t in range(t_len)], axis=1)
    stb = state_ref[j].astype(BF)
    stt = jnp.concatenate([stb] * t_len, axis=1)
    m = stt * ec
    e = jnp.dot(m, folde_ref[...], preferred_element_type=F32)
    e = e * (1.0 / scale)
    mx = jnp.max(e, axis=1, keepdims=True)
    w = jnp.exp(e - mx)
    den = jnp.sum(w, axis=1, keepdims=True)
    p = (w / den).astype(BF)
    pex = jnp.dot(p, exp12_ref[...], preferred_element_type=F32).astype(BF)
    wc = pex * ec
    wtd = jnp.dot(wc, foldh_ref[...], preferred_element_type=F32)
    return wtd.astype(BF)


def _mega_kernel(state_ref, enc_ref, inp_ref, s2_ref,
                 folde_ref, exp12_ref, foldh_ref,
                 wg_ref, bg_ref, wc_ref, bc_ref,
                 p1_ref, p1b_ref, p2_ref, p2b_ref, o_ref,
                 *, t_len, scale, in_dim, n):
    wts = [_attention(state_ref, enc_ref, folde_ref, exp12_ref, foldh_ref,
                      j, t_len, scale) for j in range(BB)]

    ibs = [inp_ref[j].astype(BF) for j in range(BB)]
    stbs = [state_ref[j].astype(BF) for j in range(BB)]
    xc = jnp.concatenate(ibs + wts + stbs, axis=1)

    chains = []
    for s in range(2):
        x1 = jnp.dot(s2_ref[s], xc, preferred_element_type=F32).astype(BF)
        x2 = jnp.dot(s2_ref[s], x1, preferred_element_type=F32).astype(BF)
        chains += [x1, x2]

    oi = lambda j: j * in_dim
    ow = lambda j: BB * in_dim + j * HID
    osn = lambda j: BB * in_dim + BB * HID + j * HID

    def trio(arr, j):
        return [arr[:, oi(j):oi(j) + in_dim],
                arr[:, ow(j):ow(j) + HID],
                arr[:, osn(j):osn(j) + HID]]

    xs = []
    for j in range(BB):
        cols = trio(xc, j)
        for ch in chains:
            cols += trio(ch, j)
        xs.append(jnp.concatenate(cols, axis=1))
    xg = jnp.concatenate(xs, axis=0)
    g = jnp.dot(xg, wg_ref[...], preferred_element_type=F32) + bg_ref[...]
    g = jax.nn.sigmoid(g)

    rss = [(g[j * n:(j + 1) * n, :HID] * state_ref[j]).astype(BF)
           for j in range(BB)]
    xr = jnp.concatenate(rss, axis=1)
    rchains = []
    for s in range(2):
        x1 = jnp.dot(s2_ref[s], xr, preferred_element_type=F32).astype(BF)
        x2 = jnp.dot(s2_ref[s], x1, preferred_element_type=F32).astype(BF)
        rchains += [x1, x2]

    xs = []
    for j in range(BB):
        cols = trio(xc, j)[:2] + [xr[:, j * HID:(j + 1) * HID]]
        for ch, rch in zip(chains, rchains):
            cols += trio(ch, j)[:2] + [rch[:, j * HID:(j + 1) * HID]]
        xs.append(jnp.concatenate(cols, axis=1))
    xcand = jnp.concatenate(xs, axis=0)
    c = jnp.dot(xcand, wc_ref[...], preferred_element_type=F32) + bc_ref[...]
    c = jnp.tanh(c)

    u = g[:, HID:]
    sts = state_ref[...].reshape(BB * n, HID)
    ns = u * sts + (1.0 - u) * c
    h1 = jnp.dot(ns.astype(BF), p1_ref[...], preferred_element_type=F32) + p1b_ref[...]
    h1 = jnp.maximum(h1, 0.0)
    o = jnp.dot(h1.astype(BF), p2_ref[...], preferred_element_type=F32) + p2b_ref[...]
    for j in range(BB):
        o_ref[j] = o[j * n:(j + 1) * n]


def kernel(support0, support1, inputs, state, encoder_outputs,
           W_gate, b_gate, W_cand, b_cand, P1, p1b, P2, p2b):
    b, n, in_dim = inputs.shape
    t_len = encoder_outputs.shape[1]
    h = state.shape[2]
    scale = math.sqrt(float(n * h))

    s2 = jnp.stack([support0, support1]).astype(BF)

    th = t_len * h
    lane = jnp.arange(th)
    folde = (lane[:, None] // h == jnp.arange(t_len)[None, :]).astype(BF)
    foldh = (lane[:, None] % h == jnp.arange(h)[None, :]).astype(BF)
    exp12 = folde.T

    in_size = in_dim + 2 * h
    nm = 2 * MAXSTEP + 1
    wg2 = W_gate.reshape(in_size, nm, 2 * h).transpose(1, 0, 2).reshape(in_size * nm, 2 * h).astype(BF)
    wc2 = W_cand.reshape(in_size, nm, h).transpose(1, 0, 2).reshape(in_size * nm, h).astype(BF)

    full = lambda shp: pl.BlockSpec(shp, lambda i: tuple(0 for _ in shp))

    out = pl.pallas_call(
        functools.partial(_mega_kernel, t_len=t_len, scale=scale,
                          in_dim=in_dim, n=n),
        grid=(b // BB,),
        in_specs=[
            pl.BlockSpec((BB, n, h), lambda i: (i, 0, 0)),
            pl.BlockSpec((BB, t_len, n, h), lambda i: (i, 0, 0, 0)),
            pl.BlockSpec((BB, n, in_dim), lambda i: (i, 0, 0)),
            full((2, n, n)),
            full((th, t_len)), full((t_len, th)), full((th, h)),
            full((in_size * nm, 2 * h)), full((1, 2 * h)),
            full((in_size * nm, h)), full((1, h)),
            full((h, h)), full((1, h)), full((h, 1)), full((1, 1)),
        ],
        out_specs=pl.BlockSpec((BB, n, 1), lambda i: (i, 0, 0)),
        out_shape=jax.ShapeDtypeStruct((b, n, 1), F32),
    )(state, encoder_outputs, inputs, s2, folde, exp12, foldh,
      wg2, b_gate.reshape(1, 2 * h), wc2, b_cand.reshape(1, h),
      P1.astype(BF), p1b.reshape(1, h), P2.astype(BF), p2b.reshape(1, 1))

    return out.reshape(b, n)
